# baseline (speedup 1.0000x reference)
import jax
import jax.numpy as jnp
from jax import lax
from jax.experimental import pallas as pl
from jax.experimental.pallas import tpu as pltpu
from jax.experimental.pallas import tpu_sc as plsc

VOCAB = 30522
HIDDEN = 768
B, S = 4, 2048
N_TOK = B * S
NW = 32
TPW = N_TOK // NW
CHUNK = 32
NCHUNK = TPW // CHUNK
LANES = 16
NV = HIDDEN // LANES


def _hsum_bcast(v):
    lane = lax.iota(jnp.int32, LANES)
    for stride in (1, 2, 4, 8):
        v = v + v.at[lane ^ stride].get(mode="promise_in_bounds")
    return v


def _sc_embed_ln(ids, tts, pos1, pos2, word_emb, pos_emb, tok_emb, gamma, beta):
    mesh = plsc.VectorSubcoreMesh(core_axis_name="c", subcore_axis_name="s")

    def body(ids_h, tts_h, pos1_h, pos2_h, word_h, pos_h, tok_h, g_h, b_h,
             out_h, idxw_v, idxp_v, idxt_v, acc_v, p_v, t_v, g_v, b_v, sem):
        wid = lax.axis_index("c") * 16 + lax.axis_index("s")
        base = wid * TPW

        pltpu.sync_copy(g_h, g_v)
        pltpu.sync_copy(b_h, b_v)

        def chunk_body(c, _):
            tb = base + c * CHUNK
            pltpu.sync_copy(ids_h.at[pl.ds(tb, CHUNK)], idxw_v)
            pltpu.sync_copy(pos1_h.at[pl.ds(tb, CHUNK)], idxp_v.at[pl.ds(0, CHUNK)])
            pltpu.sync_copy(pos2_h.at[pl.ds(tb, CHUNK)], idxp_v.at[pl.ds(CHUNK, CHUNK)])
            pltpu.sync_copy(tts_h.at[pl.ds(tb, CHUNK)], idxt_v)
            d1 = pltpu.async_copy(word_h.at[idxw_v], acc_v, sem)
            d2 = pltpu.async_copy(pos_h.at[idxp_v], p_v, sem)
            d3 = pltpu.async_copy(tok_h.at[idxt_v], t_v, sem)
            d1.wait()
            d2.wait()
            d3.wait()

            def tok_body(t, _):
                s = jnp.zeros((LANES,), jnp.float32)
                for j in range(NV):
                    sl = pl.ds(j * LANES, LANES)
                    v = (acc_v[t, sl] + p_v[t, sl] + p_v[t + CHUNK, sl]
                         + t_v[t, sl])
                    acc_v[t, sl] = v
                    s = s + v
                mv = _hsum_bcast(s) * (1.0 / HIDDEN)
                s2 = jnp.zeros((LANES,), jnp.float32)
                for j in range(NV):
                    sl = pl.ds(j * LANES, LANES)
                    d = acc_v[t, sl] - mv
                    s2 = s2 + d * d
                x = _hsum_bcast(s2) * (1.0 / HIDDEN) + 1e-12
                bits = plsc.bitcast(x, jnp.int32)
                y = plsc.bitcast(jnp.int32(0x5F3759DF) - (bits >> 1),
                                 jnp.float32)
                for _ in range(3):
                    y = y * (1.5 - 0.5 * x * y * y)
                for j in range(NV):
                    sl = pl.ds(j * LANES, LANES)
                    acc_v[t, sl] = (acc_v[t, sl] - mv) * y * g_v[sl] + b_v[sl]
                return 0

            lax.fori_loop(0, CHUNK, tok_body, 0)
            pltpu.sync_copy(acc_v, out_h.at[pl.ds(tb, CHUNK)])
            return 0

        lax.fori_loop(0, NCHUNK, chunk_body, 0)

    f = pl.kernel(
        body,
        out_type=jax.ShapeDtypeStruct((N_TOK, HIDDEN), jnp.float32),
        mesh=mesh,
        compiler_params=pltpu.CompilerParams(needs_layout_passes=False),
        scratch_types=[
            pltpu.VMEM((CHUNK,), jnp.int32),
            pltpu.VMEM((2 * CHUNK,), jnp.int32),
            pltpu.VMEM((CHUNK,), jnp.int32),
            pltpu.VMEM((CHUNK, HIDDEN), jnp.float32),
            pltpu.VMEM((2 * CHUNK, HIDDEN), jnp.float32),
            pltpu.VMEM((CHUNK, HIDDEN), jnp.float32),
            pltpu.VMEM((HIDDEN,), jnp.float32),
            pltpu.VMEM((HIDDEN,), jnp.float32),
            pltpu.SemaphoreType.DMA,
        ],
    )
    return f(ids, tts, pos1, pos2, word_emb, pos_emb, tok_emb, gamma, beta)


def kernel(input_ids, token_type_ids, position_ids, position_ids_second,
           word_emb, pos_emb, pos_emb2, tok_emb, gamma, beta):
    ids = input_ids.reshape(-1).astype(jnp.int32)
    tts = token_type_ids.reshape(-1).astype(jnp.int32)
    pos1 = position_ids.reshape(-1).astype(jnp.int32)
    pos2 = position_ids_second.reshape(-1).astype(jnp.int32)
    out = _sc_embed_ln(ids, tts, pos1, pos2, word_emb, pos_emb, tok_emb,
                       gamma, beta)
    return out.reshape(B, S, HIDDEN)


# speedup vs baseline: 1.6871x; 1.6871x over previous
import jax
import jax.numpy as jnp
from jax import lax
from jax.experimental import pallas as pl
from jax.experimental.pallas import tpu as pltpu
from jax.experimental.pallas import tpu_sc as plsc

VOCAB = 30522
HIDDEN = 768
B, S = 4, 2048
N_TOK = B * S
NW = 32
TPW = N_TOK // NW
CHUNK = 16
NCHUNK = TPW // CHUNK
NPAIR = NCHUNK // 2
LANES = 16
NV = HIDDEN // LANES


def _hsum_bcast(v):
    lane = lax.iota(jnp.int32, LANES)
    for stride in (1, 2, 4, 8):
        v = v + v.at[lane ^ stride].get(mode="promise_in_bounds")
    return v


def _sc_embed_ln(ids3, posb3, tts3, word_emb, pos_emb, tok_emb, gamma, beta):
    mesh = plsc.VectorSubcoreMesh(core_axis_name="c", subcore_axis_name="s")

    def body(ids_h, posb_h, tts_h, word_h, pos_h, tok_h, g_h, b_h, out_h,
             idw_v, idp_v, idt_v,
             w0, p0, o0, w1, p1, o1,
             tk_v, g_v, b_v, sg0, sg1, ss0, ss1):
        wid = lax.axis_index("c") * 16 + lax.axis_index("s")
        base = wid * TPW
        lane = lax.iota(jnp.int32, LANES)

        pltpu.sync_copy(tok_h, tk_v)
        pltpu.sync_copy(g_h, g_v)
        pltpu.sync_copy(b_h, b_v)
        pltpu.sync_copy(ids_h.at[wid], idw_v)
        pltpu.sync_copy(posb_h.at[wid], idp_v)
        pltpu.sync_copy(tts_h.at[wid], idt_v)

        slots = ((w0, p0, o0, sg0, ss0), (w1, p1, o1, sg1, ss1))

        def issue_gather(c, slot):
            wv, pv, ov, sg, ss = slot
            pltpu.async_copy(word_h.at[idw_v.at[c]], wv, sg)
            pltpu.async_copy(pos_h.at[idp_v.at[c]], pv, sg)

        def drain_gather(c, slot):
            wv, pv, ov, sg, ss = slot
            pltpu.make_async_copy(word_h.at[idw_v.at[c]], wv, sg).wait()
            pltpu.make_async_copy(pos_h.at[idp_v.at[c]], pv, sg).wait()

        def out_ref(c):
            return out_h.at[pl.ds(base + c * CHUNK, CHUNK)]

        def compute(c, slot):
            wv, pv, ov, sg, ss = slot
            tt16 = idt_v[c, :]

            def tok_body(ti, _):
                tt_bc = tt16.at[lax.broadcast(ti, (LANES,))].get(
                    mode="promise_in_bounds")
                sA = [jnp.zeros((LANES,), jnp.float32) for _ in range(4)]
                qA = [jnp.zeros((LANES,), jnp.float32) for _ in range(4)]
                for j in range(NV):
                    sl = pl.ds(j * LANES, LANES)
                    tok_j = plsc.load_gather(tk_v, [tt_bc, lane + j * LANES])
                    v = (wv[ti, sl] + pv[ti, sl] + pv[ti + CHUNK, sl]
                         + tok_j)
                    ov[ti, sl] = v
                    sA[j & 3] = sA[j & 3] + v
                    qA[j & 3] = qA[j & 3] + v * v
                s = (sA[0] + sA[1]) + (sA[2] + sA[3])
                q = (qA[0] + qA[1]) + (qA[2] + qA[3])
                mv = _hsum_bcast(s) * (1.0 / HIDDEN)
                ex2 = _hsum_bcast(q) * (1.0 / HIDDEN)
                x = (ex2 - mv * mv) + 1e-12
                bits = plsc.bitcast(x, jnp.int32)
                y = plsc.bitcast(jnp.int32(0x5F3759DF) - (bits >> 1),
                                 jnp.float32)
                for _ in range(3):
                    y = y * (1.5 - 0.5 * x * y * y)
                for j in range(NV):
                    sl = pl.ds(j * LANES, LANES)
                    ov[ti, sl] = (ov[ti, sl] - mv) * y * g_v[sl] + b_v[sl]
                return 0

            lax.fori_loop(0, CHUNK, tok_body, 0)

        issue_gather(0, slots[0])
        issue_gather(1, slots[1])

        def pair_body(k, _):
            for si in range(2):
                slot = slots[si]
                wv, pv, ov, sg, ss = slot
                c = 2 * k + si
                drain_gather(c, slot)

                @pl.when(k > 0)
                def _():
                    pltpu.make_async_copy(ov, out_ref(c - 2), ss).wait()

                compute(c, slot)
                pltpu.async_copy(ov, out_ref(c), ss)

                @pl.when(k < NPAIR - 1)
                def _():
                    issue_gather(c + 2, slot)
            return 0

        lax.fori_loop(0, NPAIR, pair_body, 0)
        pltpu.make_async_copy(o0, out_ref(NCHUNK - 2), ss0).wait()
        pltpu.make_async_copy(o1, out_ref(NCHUNK - 1), ss1).wait()

    f = pl.kernel(
        body,
        out_type=jax.ShapeDtypeStruct((N_TOK, HIDDEN), jnp.float32),
        mesh=mesh,
        compiler_params=pltpu.CompilerParams(needs_layout_passes=False),
        scratch_types=[
            pltpu.VMEM((NCHUNK, CHUNK), jnp.int32),
            pltpu.VMEM((NCHUNK, 2 * CHUNK), jnp.int32),
            pltpu.VMEM((NCHUNK, CHUNK), jnp.int32),
            pltpu.VMEM((CHUNK, HIDDEN), jnp.float32),
            pltpu.VMEM((2 * CHUNK, HIDDEN), jnp.float32),
            pltpu.VMEM((CHUNK, HIDDEN), jnp.float32),
            pltpu.VMEM((CHUNK, HIDDEN), jnp.float32),
            pltpu.VMEM((2 * CHUNK, HIDDEN), jnp.float32),
            pltpu.VMEM((CHUNK, HIDDEN), jnp.float32),
            pltpu.VMEM((2, HIDDEN), jnp.float32),
            pltpu.VMEM((HIDDEN,), jnp.float32),
            pltpu.VMEM((HIDDEN,), jnp.float32),
            pltpu.SemaphoreType.DMA,
            pltpu.SemaphoreType.DMA,
            pltpu.SemaphoreType.DMA,
            pltpu.SemaphoreType.DMA,
        ],
    )
    return f(ids3, posb3, tts3, word_emb, pos_emb, tok_emb, gamma, beta)


def kernel(input_ids, token_type_ids, position_ids, position_ids_second,
           word_emb, pos_emb, pos_emb2, tok_emb, gamma, beta):
    ids3 = input_ids.reshape(NW, NCHUNK, CHUNK).astype(jnp.int32)
    tts3 = token_type_ids.reshape(NW, NCHUNK, CHUNK).astype(jnp.int32)
    pA = position_ids.reshape(NW, NCHUNK, CHUNK).astype(jnp.int32)
    pB = position_ids_second.reshape(NW, NCHUNK, CHUNK).astype(jnp.int32)
    posb3 = jnp.concatenate([pA, pB], axis=-1)
    out = _sc_embed_ln(ids3, posb3, tts3, word_emb, pos_emb, tok_emb,
                       gamma, beta)
    return out.reshape(B, S, HIDDEN)
